# baseline (speedup 1.0000x reference)
import functools

import jax
import jax.numpy as jnp
from jax import lax
from jax.experimental import pallas as pl
from jax.experimental.pallas import tpu as pltpu
from jax.experimental.pallas import tpu_sc as plsc

NC = 2
NS = 16
NW = NC * NS
IDX_ROW = 128
CHUNK = 512


@functools.lru_cache(maxsize=None)
def _make_lookup(n_idx: int, vocab: int, half: int):
    assert n_idx % (NW * CHUNK) == 0 and CHUNK % IDX_ROW == 0
    b_per_w = n_idx // NW
    rows_per_w = b_per_w // IDX_ROW
    k = CHUNK // IDX_ROW
    n_chunks = b_per_w // CHUNK
    assert n_chunks % 2 == 0

    mesh = plsc.VectorSubcoreMesh(core_axis_name="c", subcore_axis_name="s")

    @functools.partial(
        pl.kernel,
        mesh=mesh,
        out_type=jax.ShapeDtypeStruct((n_idx, half), jnp.int32),
        scratch_types=[
            pltpu.VMEM((rows_per_w, IDX_ROW), jnp.int32),
            pltpu.VMEM((2, CHUNK, half), jnp.int32),
            [pltpu.SemaphoreType.DMA] * 2,
            [pltpu.SemaphoreType.DMA] * 2,
        ],
        compiler_params=pltpu.CompilerParams(use_tc_tiling_on_sc=False),
    )
    def lookup(idx_hbm, table_hbm, out_hbm, idx_v, gath_v, gsems, ssems):
        wid = lax.axis_index("s") * NC + lax.axis_index("c")
        out_base = wid * b_per_w
        pltpu.sync_copy(idx_hbm.at[pl.ds(wid * rows_per_w, rows_per_w)], idx_v)

        def fire_gather(c, b):
            for j in range(k):
                pltpu.async_copy(
                    table_hbm.at[idx_v.at[c * k + j]],
                    gath_v.at[b, pl.ds(j * IDX_ROW, IDX_ROW)],
                    gsems[b],
                )

        def wait_gather(b):
            for j in range(k):
                pltpu.make_async_copy(
                    table_hbm.at[idx_v.at[0]],
                    gath_v.at[b, pl.ds(j * IDX_ROW, IDX_ROW)],
                    gsems[b],
                ).wait()

        fire_gather(0, 0)
        fire_gather(1, 1)

        def body(q, _):
            for b in range(2):
                c = 2 * q + b
                wait_gather(b)
                pltpu.async_copy(
                    gath_v.at[b],
                    out_hbm.at[pl.ds(out_base + c * CHUNK, CHUNK)],
                    ssems[b],
                ).wait()

                @pl.when(c + 2 < n_chunks)
                def _():
                    fire_gather(c + 2, b)

            return 0

        lax.fori_loop(0, n_chunks // 2, body, 0)

    return lookup


def kernel(token_ids, weight):
    vocab, dim = weight.shape
    half = dim // 2
    ids = token_ids.reshape(-1).astype(jnp.int32)
    n_idx = ids.shape[0]
    idx2d = ids.reshape(n_idx // IDX_ROW, IDX_ROW)
    w_bf16 = weight.astype(jnp.bfloat16)
    w_packed = jax.lax.bitcast_convert_type(
        w_bf16.reshape(vocab, half, 2), jnp.int32
    )
    packed = _make_lookup(n_idx, vocab, half)(idx2d, w_packed)
    rows_bf16 = jax.lax.bitcast_convert_type(packed, jnp.bfloat16)
    out = rows_bf16.reshape(n_idx, dim).astype(jnp.float32)
    return out.reshape(token_ids.shape + (dim,))
